# baseline (speedup 1.0000x reference)
import functools
import jax
import jax.numpy as jnp
from jax import lax
from jax.experimental import pallas as pl
from jax.experimental.pallas import tpu as pltpu
from jax.experimental.pallas import tpu_sc as plsc

_N = 65536
_D = 1024
_NW = 32
_G = 16
_ROWS_PER_W = _N // _NW
_GROUPS = _ROWS_PER_W // _G
_GW = _G * _D


def _sc_body(x_hbm, o_hbm, buf, obuf):
    wid = lax.axis_index("s") * 2 + lax.axis_index("c")
    word0 = wid * (_ROWS_PER_W * _D)
    row_off = lax.iota(jnp.int32, _G) * _D

    def group_body(g, _):
        base = word0 + g * _GW
        pltpu.sync_copy(x_hbm.at[pl.ds(base, _GW)], buf)

        def col_body(cb, acc):
            c0 = cb * 8
            for u in range(8):
                idx = row_off + (c0 + u)
                v = plsc.load_gather(buf, [idx])
                acc = acc + v
                plsc.store_scatter(obuf, [idx], acc)
            return acc

        lax.fori_loop(0, _D // 8, col_body, jnp.zeros((_G,), jnp.float32))
        pltpu.sync_copy(obuf, o_hbm.at[pl.ds(base, _GW)])
        return 0

    lax.fori_loop(0, _GROUPS, group_body, 0)


@functools.partial(
    pl.kernel,
    out_type=jax.ShapeDtypeStruct((_N * _D,), jnp.float32),
    mesh=plsc.VectorSubcoreMesh(
        core_axis_name="c", subcore_axis_name="s", num_cores=2, num_subcores=16
    ),
    scratch_types=[pltpu.VMEM((_GW,), jnp.float32), pltpu.VMEM((_GW,), jnp.float32)],
    compiler_params=pltpu.CompilerParams(needs_layout_passes=False),
)
def _sc_cumsum(x_hbm, o_hbm, buf, obuf):
    _sc_body(x_hbm, o_hbm, buf, obuf)


def kernel(x):
    n, d = x.shape
    return _sc_cumsum(x.reshape(-1)).reshape(n, d)


# speedup vs baseline: 3.6317x; 3.6317x over previous
import functools
import jax
import jax.numpy as jnp
from jax import lax
from jax.experimental import pallas as pl
from jax.experimental.pallas import tpu as pltpu
from jax.experimental.pallas import tpu_sc as plsc

_N = 65536
_D = 1024
_NW = 32
_G = 16
_ROWS_PER_W = _N // _NW
_GROUPS = _ROWS_PER_W // _G
_GW = _G * _D
_L = 16
_BLOCKS = _D // _L


def _sc_body(x_hbm, o_hbm, buf, obuf):
    last = jnp.full((_L,), _L - 1, dtype=jnp.int32)
    wid = lax.axis_index("s") * 2 + lax.axis_index("c")
    word0 = wid * (_ROWS_PER_W * _D)

    def group_body(g, _):
        base = word0 + g * _GW
        pltpu.sync_copy(x_hbm.at[pl.ds(base, _GW)], buf)

        def blk_body(cb, carries):
            col = cb * _L
            new = []
            for r in range(_G):
                off = r * _D + col
                v = buf[pl.ds(off, _L)]
                s = plsc.cumsum(v) + carries[r]
                obuf[pl.ds(off, _L)] = s
                new.append(jnp.take(s, last))
            return tuple(new)

        zero = jnp.zeros((_L,), jnp.float32)
        lax.fori_loop(0, _BLOCKS, blk_body, (zero,) * _G)
        pltpu.sync_copy(obuf, o_hbm.at[pl.ds(base, _GW)])
        return 0

    lax.fori_loop(0, _GROUPS, group_body, 0)


@functools.partial(
    pl.kernel,
    out_type=jax.ShapeDtypeStruct((_N * _D,), jnp.float32),
    mesh=plsc.VectorSubcoreMesh(
        core_axis_name="c", subcore_axis_name="s", num_cores=2, num_subcores=16
    ),
    scratch_types=[pltpu.VMEM((_GW,), jnp.float32), pltpu.VMEM((_GW,), jnp.float32)],
    compiler_params=pltpu.CompilerParams(needs_layout_passes=False),
)
def _sc_cumsum(x_hbm, o_hbm, buf, obuf):
    _sc_body(x_hbm, o_hbm, buf, obuf)


def kernel(x):
    n, d = x.shape
    return _sc_cumsum(x.reshape(-1)).reshape(n, d)


# speedup vs baseline: 19.4161x; 5.3462x over previous
import jax
import jax.numpy as jnp
from jax.experimental import pallas as pl

_CHUNK = 128


def _cumsum_kernel(x_ref, tri_ref, o_ref):
    tri = tri_ref[...]
    nchunks = x_ref.shape[1] // _CHUNK
    carry = jnp.zeros((x_ref.shape[0], 1), dtype=jnp.float32)
    for k in range(nchunks):
        sl = pl.ds(k * _CHUNK, _CHUNK)
        chunk = x_ref[:, sl]
        within = jax.lax.dot(chunk, tri, preferred_element_type=jnp.float32)
        out = within + carry
        o_ref[:, sl] = out
        carry = out[:, _CHUNK - 1:_CHUNK]


def kernel(x):
    n, d = x.shape
    rb = 2048
    tri = jnp.triu(jnp.ones((_CHUNK, _CHUNK), dtype=jnp.float32))
    return pl.pallas_call(
        _cumsum_kernel,
        grid=(n // rb,),
        in_specs=[
            pl.BlockSpec((rb, d), lambda i: (i, 0)),
            pl.BlockSpec((_CHUNK, _CHUNK), lambda i: (0, 0)),
        ],
        out_specs=pl.BlockSpec((rb, d), lambda i: (i, 0)),
        out_shape=jax.ShapeDtypeStruct((n, d), jnp.float32),
    )(x, tri)
